# baseline (speedup 1.0000x reference)
import functools

import jax
import jax.numpy as jnp
from jax import lax
from jax.experimental import pallas as pl
from jax.experimental.pallas import tpu as pltpu
from jax.experimental.pallas import tpu_sc as plsc

EMBED_DIM = 64
BATCH = 4096
DOC_P = 200
QRY_P = 20

NW = 32
BSLICE = BATCH // NW


def _body(doc_t, qry_t, table, out_doc, out_qry, idxb, gbuf, obuf, gsems):
    wid = lax.axis_index("s") * 2 + lax.axis_index("c")
    bcol = wid * BSLICE
    lanes = lax.iota(jnp.int32, 16)

    def run_phase(idx_t, out_t, npos):
        def start(p, par):
            pltpu.sync_copy(idx_t.at[p, pl.ds(bcol, BSLICE)], idxb.at[par])
            pltpu.async_copy(table.at[idxb.at[par]], gbuf.at[par],
                             gsems.at[par])

        def consume(p, par):
            pltpu.make_async_copy(
                table.at[pl.ds(0, BSLICE)], gbuf.at[par],
                gsems.at[par]).wait()

            def col(d, carry):
                dl = jnp.full((16,), 0, jnp.int32) + d
                for k in range(BSLICE // 16):
                    vals = plsc.load_gather(
                        gbuf, [jnp.full((16,), par, jnp.int32),
                               k * 16 + lanes, dl])
                    obuf[d, pl.ds(k * 16, 16)] = vals
                return carry

            lax.fori_loop(0, EMBED_DIM, col, 0)
            pltpu.sync_copy(obuf, out_t.at[p, :, pl.ds(bcol, BSLICE)])

        start(0, 0)

        def pair(g, carry):
            for par in range(2):
                p = g * 2 + par

                @pl.when(p + 1 < npos)
                def _():
                    start(p + 1, 1 - par)

                consume(p, par)
            return carry

        lax.fori_loop(0, npos // 2, pair, 0)

    run_phase(doc_t, out_doc, DOC_P)
    run_phase(qry_t, out_qry, QRY_P)


@jax.jit
def _embed(doc_t, qry_t, table):
    mesh = plsc.VectorSubcoreMesh(core_axis_name="c", subcore_axis_name="s")
    run = functools.partial(
        pl.kernel,
        mesh=mesh,
        compiler_params=pltpu.CompilerParams(use_tc_tiling_on_sc=False,
                                             needs_layout_passes=False),
        out_type=[
            jax.ShapeDtypeStruct((DOC_P, EMBED_DIM, BATCH), jnp.float32),
            jax.ShapeDtypeStruct((QRY_P, EMBED_DIM, BATCH), jnp.float32),
        ],
        scratch_types=[
            pltpu.VMEM((2, BSLICE), jnp.int32),
            pltpu.VMEM((2, BSLICE, EMBED_DIM), jnp.float32),
            pltpu.VMEM((EMBED_DIM, BSLICE), jnp.float32),
            pltpu.SemaphoreType.DMA((2,)),
        ],
    )(_body)
    return run(doc_t, qry_t, table)


def kernel(doc, qry, table):
    out_doc_t, out_qry_t = _embed(doc.T, qry.T, table)
    return (out_doc_t.transpose(2, 0, 1), out_qry_t.transpose(2, 0, 1))


# speedup vs baseline: 1.8284x; 1.8284x over previous
import functools

import jax
import jax.numpy as jnp
from jax import lax
from jax.experimental import pallas as pl
from jax.experimental.pallas import tpu as pltpu
from jax.experimental.pallas import tpu_sc as plsc

EMBED_DIM = 64
CHUNK = 128
NBUF = 10

DOC_TOK = 4096 * 200
QRY_TOK = 4096 * 20

NW = 32
DOC_PER_W = DOC_TOK // NW
QRY_PER_W = QRY_TOK // NW
DOC_STEPS = DOC_PER_W // CHUNK
QRY_STEPS = QRY_PER_W // CHUNK
DOC_GROUPS = DOC_STEPS // NBUF
QRY_GROUPS = QRY_STEPS // NBUF


def _body(doc_idx, qry_idx, table, out_doc, out_qry,
          idx_d, idx_q, bufs, gsems, wsems):
    wid = lax.axis_index("s") * 2 + lax.axis_index("c")

    pltpu.sync_copy(doc_idx.at[pl.ds(wid * DOC_PER_W, DOC_PER_W)], idx_d)
    pltpu.sync_copy(qry_idx.at[pl.ds(wid * QRY_PER_W, QRY_PER_W)], idx_q)

    def run_phase(idx_ref, out_ref, base_tok, ngroups):
        def gather(c, b):
            pltpu.async_copy(
                table.at[idx_ref.at[pl.ds(c * CHUNK, CHUNK)]],
                bufs.at[b], gsems.at[b])

        def write(c, b):
            pltpu.async_copy(
                bufs.at[b],
                out_ref.at[pl.ds(base_tok + c * CHUNK, CHUNK)],
                wsems.at[b])

        for b in range(NBUF):
            gather(b, b)

        def group_body(g, carry):
            base = g * NBUF
            for b in range(NBUF):
                pltpu.make_async_copy(
                    table.at[pl.ds(0, CHUNK)],
                    bufs.at[b], gsems.at[b]).wait()
                write(base + b, b)
            @pl.when(g < ngroups - 1)
            def _():
                for b in range(NBUF):
                    pltpu.make_async_copy(
                        bufs.at[b],
                        out_ref.at[pl.ds(base_tok, CHUNK)],
                        wsems.at[b]).wait()
                    gather(base + NBUF + b, b)
            return carry

        lax.fori_loop(0, ngroups, group_body, 0)
        for b in range(NBUF):
            pltpu.make_async_copy(
                bufs.at[b],
                out_ref.at[pl.ds(base_tok, CHUNK)],
                wsems.at[b]).wait()

    run_phase(idx_d, out_doc, wid * DOC_PER_W, DOC_GROUPS)
    run_phase(idx_q, out_qry, wid * QRY_PER_W, QRY_GROUPS)


@jax.jit
def _embed(doc_idx, qry_idx, table):
    mesh = plsc.VectorSubcoreMesh(core_axis_name="c", subcore_axis_name="s")
    run = functools.partial(
        pl.kernel,
        mesh=mesh,
        compiler_params=pltpu.CompilerParams(use_tc_tiling_on_sc=False,
                                             skip_device_barrier=True),
        out_type=[
            jax.ShapeDtypeStruct((DOC_TOK, EMBED_DIM), jnp.float32),
            jax.ShapeDtypeStruct((QRY_TOK, EMBED_DIM), jnp.float32),
        ],
        scratch_types=[
            pltpu.VMEM((DOC_PER_W,), jnp.int32),
            pltpu.VMEM((QRY_PER_W,), jnp.int32),
            pltpu.VMEM((NBUF, CHUNK, EMBED_DIM), jnp.float32),
            pltpu.SemaphoreType.DMA((NBUF,)),
            pltpu.SemaphoreType.DMA((NBUF,)),
        ],
    )(_body)
    return run(doc_idx, qry_idx, table)


def kernel(doc, qry, table):
    doc_idx = doc.reshape(DOC_TOK)
    qry_idx = qry.reshape(QRY_TOK)
    out_doc, out_qry = _embed(doc_idx, qry_idx, table)
    return (out_doc.reshape(*doc.shape, EMBED_DIM),
            out_qry.reshape(*qry.shape, EMBED_DIM))


# speedup vs baseline: 2.3490x; 1.2847x over previous
import functools

import jax
import jax.numpy as jnp
from jax import lax
from jax.experimental import pallas as pl
from jax.experimental.pallas import tpu as pltpu
from jax.experimental.pallas import tpu_sc as plsc

EMBED_DIM = 64
CHUNK = 128
NBUF = 10

DOC_TOK = 4096 * 200
QRY_TOK = 4096 * 20

NW = 32
DOC_PER_W = DOC_TOK // NW
QRY_PER_W = QRY_TOK // NW
DOC_STEPS = DOC_PER_W // CHUNK
QRY_STEPS = QRY_PER_W // CHUNK
DOC_GROUPS = DOC_STEPS // NBUF
QRY_GROUPS = QRY_STEPS // NBUF


def _body(doc_idx, qry_idx, table, out_doc, out_qry,
          idx_d, idx_q, bufs, gsems, wsems):
    wid = lax.axis_index("s") * 2 + lax.axis_index("c")

    pltpu.sync_copy(doc_idx.at[pl.ds(wid * DOC_PER_W, DOC_PER_W)], idx_d)
    pltpu.sync_copy(qry_idx.at[pl.ds(wid * QRY_PER_W, QRY_PER_W)], idx_q)

    def run_phase(idx_ref, out_ref, base_tok, ngroups):
        def gather(c, b):
            pltpu.async_copy(
                table.at[idx_ref.at[pl.ds(c * CHUNK, CHUNK)]],
                bufs.at[b], gsems.at[b])

        def write(c, b):
            pltpu.async_copy(
                bufs.at[b],
                out_ref.at[pl.ds(base_tok + c * CHUNK, CHUNK),
                           pl.ds(0, EMBED_DIM)],
                wsems.at[b])

        for b in range(NBUF):
            gather(b, b)

        def group_body(g, carry):
            base = g * NBUF
            for b in range(NBUF):
                pltpu.make_async_copy(
                    table.at[pl.ds(0, CHUNK)],
                    bufs.at[b], gsems.at[b]).wait()
                write(base + b, b)
            @pl.when(g < ngroups - 1)
            def _():
                for b in range(NBUF):
                    pltpu.make_async_copy(
                        bufs.at[b],
                        out_ref.at[pl.ds(base_tok, CHUNK),
                                   pl.ds(0, EMBED_DIM)],
                        wsems.at[b]).wait()
                    gather(base + NBUF + b, b)
            return carry

        lax.fori_loop(0, ngroups, group_body, 0)
        for b in range(NBUF):
            pltpu.make_async_copy(
                bufs.at[b],
                out_ref.at[pl.ds(base_tok, CHUNK), pl.ds(0, EMBED_DIM)],
                wsems.at[b]).wait()

    run_phase(idx_d, out_doc, wid * DOC_PER_W, DOC_GROUPS)
    run_phase(idx_q, out_qry, wid * QRY_PER_W, QRY_GROUPS)


@jax.jit
def _embed(doc_idx, qry_idx, table):
    mesh = plsc.VectorSubcoreMesh(core_axis_name="c", subcore_axis_name="s")
    run = functools.partial(
        pl.kernel,
        mesh=mesh,
        compiler_params=pltpu.CompilerParams(use_tc_tiling_on_sc=False,
                                             skip_device_barrier=True),
        out_type=[
            jax.ShapeDtypeStruct((DOC_TOK, 2 * EMBED_DIM), jnp.float32),
            jax.ShapeDtypeStruct((QRY_TOK, 2 * EMBED_DIM), jnp.float32),
        ],
        scratch_types=[
            pltpu.VMEM((DOC_PER_W,), jnp.int32),
            pltpu.VMEM((QRY_PER_W,), jnp.int32),
            pltpu.VMEM((NBUF, CHUNK, EMBED_DIM), jnp.float32),
            pltpu.SemaphoreType.DMA((NBUF,)),
            pltpu.SemaphoreType.DMA((NBUF,)),
        ],
    )(_body)
    return run(doc_idx, qry_idx, table)


def kernel(doc, qry, table):
    doc_idx = doc.reshape(DOC_TOK)
    qry_idx = qry.reshape(QRY_TOK)
    out_doc, out_qry = _embed(doc_idx, qry_idx, table)
    out_doc = out_doc.reshape(*doc.shape, 2 * EMBED_DIM)[:, :, :EMBED_DIM]
    out_qry = out_qry.reshape(*qry.shape, 2 * EMBED_DIM)[:, :, :EMBED_DIM]
    return (out_doc, out_qry)
